# baseline (speedup 1.0000x reference)
import jax
import jax.numpy as jnp
from jax import lax
from jax.experimental import pallas as pl
from jax.experimental.pallas import tpu as pltpu
from jax.experimental.pallas import tpu_sc as plsc

NUM_CORES = 2
NUM_SUBCORES = 16
LANES = 16
NW = NUM_CORES * NUM_SUBCORES

BATCH = 16384
B_PER_W = BATCH // NW
CHUNK = 128
NCHUNK = B_PER_W // CHUNK
D = 16


def _mf_body(u_hbm, i_hbm, utT_hbm, itT_hbm, out_hbm, *scratch):
    idx_u, idx_i = scratch[0], scratch[1]
    ucols = scratch[2:2 + D]
    icols = scratch[2 + D:2 + 2 * D]
    out_v = scratch[2 + 2 * D]
    sem = scratch[3 + 2 * D]

    cid = lax.axis_index("c")
    sid = lax.axis_index("s")
    wid = sid * NUM_CORES + cid
    base = wid * B_PER_W

    pltpu.sync_copy(u_hbm.at[pl.ds(base, B_PER_W)], idx_u)
    pltpu.sync_copy(i_hbm.at[pl.ds(base, B_PER_W)], idx_i)

    for c in range(NCHUNK):
        cbase = c * CHUNK
        copies = []
        for f in range(D):
            copies.append(pltpu.async_copy(
                utT_hbm.at[f].at[idx_u.at[pl.ds(cbase, CHUNK)]],
                ucols[f], sem))
            copies.append(pltpu.async_copy(
                itT_hbm.at[f].at[idx_i.at[pl.ds(cbase, CHUNK)]],
                icols[f], sem))
        for cp in copies:
            cp.wait()

        def block(b, _, cbase=cbase):
            acc = jnp.zeros((LANES,), jnp.float32)
            for f in range(D):
                acc = acc + (ucols[f][pl.ds(b * LANES, LANES)]
                             * icols[f][pl.ds(b * LANES, LANES)])
            pred = 1.0 / (1.0 + jnp.exp(-acc))
            out_v[pl.ds(cbase + b * LANES, LANES)] = pred
            return _

        lax.fori_loop(0, CHUNK // LANES, block, 0)

    pltpu.sync_copy(out_v, out_hbm.at[pl.ds(base, B_PER_W)])


@jax.jit
def _mf(u, i, user_table_t, item_table_t):
    mesh = plsc.VectorSubcoreMesh(core_axis_name="c", subcore_axis_name="s")
    scratch = (
        [pltpu.VMEM((B_PER_W,), jnp.int32) for _ in range(2)]
        + [pltpu.VMEM((CHUNK,), jnp.float32) for _ in range(2 * D)]
        + [pltpu.VMEM((B_PER_W,), jnp.float32), pltpu.SemaphoreType.DMA]
    )
    run = pl.kernel(
        _mf_body,
        out_type=jax.ShapeDtypeStruct((BATCH,), jnp.float32),
        mesh=mesh,
        scratch_types=scratch,
        compiler_params=pltpu.CompilerParams(
            needs_layout_passes=False, use_tc_tiling_on_sc=False),
    )
    return run(u, i, user_table_t, item_table_t)


def kernel(u, i, user_table, item_table):
    return _mf(u.astype(jnp.int32), i.astype(jnp.int32),
               user_table.T, item_table.T)
